# baseline (speedup 1.0000x reference)
import functools

import jax
import jax.numpy as jnp
from jax import lax
from jax.experimental import pallas as pl
from jax.experimental.pallas import tpu as pltpu
from jax.experimental.pallas import tpu_sc as plsc

N_NODES = 10000
N_EDGES = 160000
NODE_DIM = 256
EDGE_DIM = 16
HIDDEN = 512

NC = 2
NS = 16
NW = NC * NS

SLICES = 5
SLICE_E = N_EDGES // SLICES

GATHER_CHUNK = 200
SCAT_CHUNK = 200
ROWS_PER_TILE = 624
ROWS_TAIL = N_NODES - ROWS_PER_TILE * NS
HALF_DIM = NODE_DIM // 2

_sc_mesh = plsc.VectorSubcoreMesh(core_axis_name="c", subcore_axis_name="s")


@functools.cache
def _make_gather(n_edges):
    epw = n_edges // NW
    iters = epw // GATHER_CHUNK

    @functools.partial(
        pl.kernel,
        out_type=(
            jax.ShapeDtypeStruct((n_edges, HALF_DIM), jnp.int32),
            jax.ShapeDtypeStruct((n_edges, HALF_DIM), jnp.int32),
        ),
        mesh=_sc_mesh,
        scratch_types=[
            pltpu.VMEM((GATHER_CHUNK,), jnp.int32),
            pltpu.VMEM((GATHER_CHUNK,), jnp.int32),
            pltpu.VMEM((GATHER_CHUNK, HALF_DIM), jnp.int32),
            pltpu.VMEM((GATHER_CHUNK, HALF_DIM), jnp.int32),
            pltpu.SemaphoreType.DMA,
            pltpu.SemaphoreType.DMA,
        ],
    )
    def _sc_gather(h_hbm, row_hbm, col_hbm, hrow_hbm, hcol_hbm,
                   idxr_v, idxc_v, bufr_v, bufc_v, sem_r, sem_c):
        wid = lax.axis_index("s") * NC + lax.axis_index("c")
        base = wid * epw

        def body(i, carry):
            off = base + i * GATHER_CHUNK
            pltpu.sync_copy(row_hbm.at[pl.ds(off, GATHER_CHUNK)], idxr_v)
            pltpu.sync_copy(col_hbm.at[pl.ds(off, GATHER_CHUNK)], idxc_v)
            cp_r = pltpu.async_copy(h_hbm.at[idxr_v], bufr_v, sem_r)
            cp_c = pltpu.async_copy(h_hbm.at[idxc_v], bufc_v, sem_c)
            cp_r.wait()
            cp_c.wait()
            pltpu.sync_copy(bufr_v, hrow_hbm.at[pl.ds(off, GATHER_CHUNK)])
            pltpu.sync_copy(bufc_v, hcol_hbm.at[pl.ds(off, GATHER_CHUNK)])
            return carry

        lax.fori_loop(0, iters, body, 0)

    return _sc_gather


@functools.cache
def _make_scatter(n_edges):
    ept = n_edges // NS
    iters = ept // SCAT_CHUNK

    @functools.partial(
        pl.kernel,
        out_type=jax.ShapeDtypeStruct((N_NODES, NODE_DIM), jnp.float32),
        mesh=_sc_mesh,
        scratch_types=[
            pltpu.VMEM((SCAT_CHUNK,), jnp.int32),
            pltpu.VMEM((SCAT_CHUNK, HALF_DIM), jnp.float32),
            pltpu.VMEM_SHARED((N_NODES, HALF_DIM), jnp.float32),
        ],
    )
    def _sc_scatter(h_hbm, row_hbm, dh_hbm, out_hbm, idx_v, buf_v, acc_sh):
        c = lax.axis_index("c")
        s = lax.axis_index("s")
        colbase = c * HALF_DIM
        rowbase = s * ROWS_PER_TILE

        pltpu.sync_copy(
            h_hbm.at[pl.ds(rowbase, ROWS_PER_TILE), pl.ds(colbase, HALF_DIM)],
            acc_sh.at[pl.ds(rowbase, ROWS_PER_TILE)],
        )

        @pl.when(s == NS - 1)
        def _init_tail():
            tail = ROWS_PER_TILE * NS
            pltpu.sync_copy(
                h_hbm.at[pl.ds(tail, ROWS_TAIL), pl.ds(colbase, HALF_DIM)],
                acc_sh.at[pl.ds(tail, ROWS_TAIL)],
            )

        plsc.subcore_barrier()

        def body(i, carry):
            off = s * ept + i * SCAT_CHUNK
            pltpu.sync_copy(row_hbm.at[pl.ds(off, SCAT_CHUNK)], idx_v)
            pltpu.sync_copy(
                dh_hbm.at[pl.ds(off, SCAT_CHUNK), pl.ds(colbase, HALF_DIM)],
                buf_v)
            pltpu.sync_copy(buf_v, acc_sh.at[idx_v], add=True)
            return carry

        lax.fori_loop(0, iters, body, 0)
        plsc.subcore_barrier()

        pltpu.sync_copy(
            acc_sh.at[pl.ds(rowbase, ROWS_PER_TILE)],
            out_hbm.at[pl.ds(rowbase, ROWS_PER_TILE), pl.ds(colbase, HALF_DIM)],
        )

        @pl.when(s == NS - 1)
        def _write_tail():
            tail = ROWS_PER_TILE * NS
            pltpu.sync_copy(
                acc_sh.at[pl.ds(tail, ROWS_TAIL)],
                out_hbm.at[pl.ds(tail, ROWS_TAIL), pl.ds(colbase, HALF_DIM)],
            )

    return _sc_scatter


EDGE_BLOCK = 3200


def _unpack_bf16_pair(w):
    wu = jax.lax.bitcast_convert_type(w, jnp.uint32)
    lo = jax.lax.bitcast_convert_type(wu << 16, jnp.float32)
    hi = jax.lax.bitcast_convert_type(wu & jnp.uint32(0xFFFF0000), jnp.float32)
    return lo.astype(jnp.bfloat16), hi.astype(jnp.bfloat16)


def _mlp_body(hr_ref, hc_ref, ea_ref,
              W1hP_ref, W1e_ref, b1_ref, W2_ref, b2_ref, W3_ref, b3_ref,
              We1rP_ref, We1cP_ref, We1e_ref, be1_ref, We2_ref, be2_ref,
              dh_ref, eo_ref):
    f32 = jnp.float32
    bf16 = jnp.bfloat16
    hr_e, hr_o = _unpack_bf16_pair(hr_ref[...])
    hc_e, hc_o = _unpack_bf16_pair(hc_ref[...])
    hr = jnp.concatenate([hr_e, hr_o], axis=1)
    hc = jnp.concatenate([hc_e, hc_o], axis=1)
    ea = ea_ref[...]
    ea16 = ea.astype(bf16)
    x = (jnp.dot(hr, W1hP_ref[...], preferred_element_type=f32)
         + jnp.dot(ea16, W1e_ref[...], preferred_element_type=f32)
         + b1_ref[...])
    x = jnp.maximum(x, 0.0).astype(bf16)
    x = jnp.dot(x, W2_ref[...], preferred_element_type=f32) + b2_ref[...]
    x = jnp.maximum(x, 0.0).astype(bf16)
    dh_ref[...] = jnp.dot(x, W3_ref[...], preferred_element_type=f32) + b3_ref[...]
    y = (jnp.dot(hr, We1rP_ref[...], preferred_element_type=f32)
         + jnp.dot(hc, We1cP_ref[...], preferred_element_type=f32)
         + jnp.dot(ea16, We1e_ref[...], preferred_element_type=f32)
         + be1_ref[...])
    y = jnp.maximum(y, 0.0).astype(bf16)
    eo_ref[...] = (ea + jnp.dot(y, We2_ref[...], preferred_element_type=f32)
                   + be2_ref[...])


def _edge_blk(i):
    return (i, 0)


def _full(i):
    return (0, 0)


def _tc_mlp(h_row, h_col, edge_attr, *weights):
    n_edges = h_row.shape[0]
    eb = EDGE_BLOCK
    in_specs = [
        pl.BlockSpec((eb, HALF_DIM), _edge_blk),
        pl.BlockSpec((eb, HALF_DIM), _edge_blk),
        pl.BlockSpec((eb, EDGE_DIM), _edge_blk),
    ] + [pl.BlockSpec(w.shape, _full) for w in weights]
    out_specs = (
        pl.BlockSpec((eb, NODE_DIM), _edge_blk),
        pl.BlockSpec((eb, EDGE_DIM), _edge_blk),
    )
    return pl.pallas_call(
        _mlp_body,
        grid=(n_edges // eb,),
        in_specs=in_specs,
        out_specs=out_specs,
        out_shape=(
            jax.ShapeDtypeStruct((n_edges, NODE_DIM), jnp.float32),
            jax.ShapeDtypeStruct((n_edges, EDGE_DIM), jnp.float32),
        ),
    )(h_row, h_col, edge_attr, *weights)


def kernel(h, edge_index, edge_attr, W1, b1, W2, b2, W3, b3,
           We1, be1, We2, be2):
    row = edge_index[0].astype(jnp.int32)
    col = edge_index[1].astype(jnp.int32)

    bf16 = jnp.bfloat16

    def _perm(W):
        return jnp.concatenate([W[0::2], W[1::2]], axis=0).astype(bf16)

    W1hP = _perm(W1[:NODE_DIM])
    W1e = W1[NODE_DIM:].astype(bf16)
    W2 = W2.astype(bf16)
    W3 = W3.astype(bf16)
    We1rP = _perm(We1[:NODE_DIM])
    We1cP = _perm(We1[NODE_DIM:2 * NODE_DIM])
    We1e = We1[2 * NODE_DIM:].astype(bf16)
    We2 = We2.astype(bf16)
    b1r = b1.reshape(1, -1)
    b2r = b2.reshape(1, -1)
    b3r = b3.reshape(1, -1)
    be1r = be1.reshape(1, -1)
    be2r = be2.reshape(1, -1)

    h_pack = jax.lax.bitcast_convert_type(
        h.astype(bf16).reshape(N_NODES, HALF_DIM, 2), jnp.int32)

    gather = _make_gather(SLICE_E)
    scatter = _make_scatter(SLICE_E)

    h_cur = h
    eo_parts = []
    for k in range(SLICES):
        sl = slice(k * SLICE_E, (k + 1) * SLICE_E)
        row_k = row[sl]
        col_k = col[sl]
        hr_k, hc_k = gather(h_pack, row_k, col_k)
        dh_k, eo_k = _tc_mlp(hr_k, hc_k, edge_attr[sl],
                             W1hP, W1e, b1r, W2, b2r, W3, b3r,
                             We1rP, We1cP, We1e, be1r, We2, be2r)
        eo_parts.append(eo_k)
        h_cur = scatter(h_cur, row_k, dh_k)

    edge_attr_new = jnp.concatenate(eo_parts, axis=0)
    return (h_cur, edge_attr_new)


# speedup vs baseline: 1.1316x; 1.1316x over previous
import functools

import jax
import jax.numpy as jnp
from jax import lax
from jax.experimental import pallas as pl
from jax.experimental.pallas import tpu as pltpu
from jax.experimental.pallas import tpu_sc as plsc

N_NODES = 10000
N_EDGES = 160000
NODE_DIM = 256
EDGE_DIM = 16
HIDDEN = 512

NC = 2
NS = 16
NW = NC * NS

SLICES = 5
SLICE_E = N_EDGES // SLICES

GATHER_CHUNK = 200
SCAT_CHUNK = 200
ROWS_PER_TILE = 624
ROWS_TAIL = N_NODES - ROWS_PER_TILE * NS
HALF_DIM = NODE_DIM // 2

_sc_mesh = plsc.VectorSubcoreMesh(core_axis_name="c", subcore_axis_name="s")


@functools.cache
def _make_gather(n_edges):
    epw = n_edges // NW
    iters = epw // GATHER_CHUNK

    @functools.partial(
        pl.kernel,
        out_type=(
            jax.ShapeDtypeStruct((n_edges, HALF_DIM), jnp.int32),
            jax.ShapeDtypeStruct((n_edges, HALF_DIM), jnp.int32),
        ),
        mesh=_sc_mesh,
        scratch_types=[
            pltpu.VMEM((GATHER_CHUNK,), jnp.int32),
            pltpu.VMEM((GATHER_CHUNK,), jnp.int32),
            pltpu.VMEM((GATHER_CHUNK, HALF_DIM), jnp.int32),
            pltpu.VMEM((GATHER_CHUNK, HALF_DIM), jnp.int32),
            pltpu.SemaphoreType.DMA,
            pltpu.SemaphoreType.DMA,
        ],
    )
    def _sc_gather(h_hbm, row_hbm, col_hbm, hrow_hbm, hcol_hbm,
                   idxr_v, idxc_v, bufr_v, bufc_v, sem_r, sem_c):
        wid = lax.axis_index("s") * NC + lax.axis_index("c")
        base = wid * epw

        def body(i, carry):
            off = base + i * GATHER_CHUNK
            pltpu.sync_copy(row_hbm.at[pl.ds(off, GATHER_CHUNK)], idxr_v)
            pltpu.sync_copy(col_hbm.at[pl.ds(off, GATHER_CHUNK)], idxc_v)
            cp_r = pltpu.async_copy(h_hbm.at[idxr_v], bufr_v, sem_r)
            cp_c = pltpu.async_copy(h_hbm.at[idxc_v], bufc_v, sem_c)
            cp_r.wait()
            cp_c.wait()
            pltpu.sync_copy(bufr_v, hrow_hbm.at[pl.ds(off, GATHER_CHUNK)])
            pltpu.sync_copy(bufc_v, hcol_hbm.at[pl.ds(off, GATHER_CHUNK)])
            return carry

        lax.fori_loop(0, iters, body, 0)

    return _sc_gather


@functools.cache
def _make_scatter(n_edges):
    ept = n_edges // NS
    iters = ept // SCAT_CHUNK

    @functools.partial(
        pl.kernel,
        out_type=jax.ShapeDtypeStruct((N_NODES, NODE_DIM), jnp.float32),
        mesh=_sc_mesh,
        scratch_types=[
            pltpu.VMEM((SCAT_CHUNK,), jnp.int32),
            pltpu.VMEM((SCAT_CHUNK, HALF_DIM), jnp.float32),
            pltpu.VMEM_SHARED((N_NODES, HALF_DIM), jnp.float32),
        ],
    )
    def _sc_scatter(h_hbm, row_hbm, dh_hbm, out_hbm, idx_v, buf_v, acc_sh):
        c = lax.axis_index("c")
        s = lax.axis_index("s")
        colbase = c * HALF_DIM
        rowbase = s * ROWS_PER_TILE

        pltpu.sync_copy(
            h_hbm.at[pl.ds(rowbase, ROWS_PER_TILE), pl.ds(colbase, HALF_DIM)],
            acc_sh.at[pl.ds(rowbase, ROWS_PER_TILE)],
        )

        @pl.when(s == NS - 1)
        def _init_tail():
            tail = ROWS_PER_TILE * NS
            pltpu.sync_copy(
                h_hbm.at[pl.ds(tail, ROWS_TAIL), pl.ds(colbase, HALF_DIM)],
                acc_sh.at[pl.ds(tail, ROWS_TAIL)],
            )

        plsc.subcore_barrier()

        def body(i, carry):
            off = s * ept + i * SCAT_CHUNK
            pltpu.sync_copy(row_hbm.at[pl.ds(off, SCAT_CHUNK)], idx_v)
            pltpu.sync_copy(
                dh_hbm.at[pl.ds(off, SCAT_CHUNK), pl.ds(colbase, HALF_DIM)],
                buf_v)
            pltpu.sync_copy(buf_v, acc_sh.at[idx_v], add=True)
            return carry

        lax.fori_loop(0, iters, body, 0)
        plsc.subcore_barrier()

        pltpu.sync_copy(
            acc_sh.at[pl.ds(rowbase, ROWS_PER_TILE)],
            out_hbm.at[pl.ds(rowbase, ROWS_PER_TILE), pl.ds(colbase, HALF_DIM)],
        )

        @pl.when(s == NS - 1)
        def _write_tail():
            tail = ROWS_PER_TILE * NS
            pltpu.sync_copy(
                acc_sh.at[pl.ds(tail, ROWS_TAIL)],
                out_hbm.at[pl.ds(tail, ROWS_TAIL), pl.ds(colbase, HALF_DIM)],
            )

    return _sc_scatter


EDGE_BLOCK = 3200


def _unpack_bf16_pair(w):
    wu = jax.lax.bitcast_convert_type(w, jnp.uint32)
    lo = jax.lax.bitcast_convert_type(wu << 16, jnp.float32)
    hi = jax.lax.bitcast_convert_type(wu & jnp.uint32(0xFFFF0000), jnp.float32)
    return lo.astype(jnp.bfloat16), hi.astype(jnp.bfloat16)


def _mlp_body(hr_ref, hc_ref, ea_ref,
              W1hP_ref, W1e_ref, b1_ref, W2_ref, b2_ref, W3_ref, b3_ref,
              We1rP_ref, We1cP_ref, We1e_ref, be1_ref, We2_ref, be2_ref,
              dh_ref, eo_ref):
    f32 = jnp.float32
    bf16 = jnp.bfloat16
    hr_e, hr_o = _unpack_bf16_pair(hr_ref[...])
    hc_e, hc_o = _unpack_bf16_pair(hc_ref[...])
    hr = jnp.concatenate([hr_e, hr_o], axis=1)
    hc = jnp.concatenate([hc_e, hc_o], axis=1)
    ea = ea_ref[...]
    ea16 = ea.astype(bf16)
    x = (jnp.dot(hr, W1hP_ref[...], preferred_element_type=f32)
         + jnp.dot(ea16, W1e_ref[...], preferred_element_type=f32)
         + b1_ref[...])
    x = jnp.maximum(x, 0.0).astype(bf16)
    x = jnp.dot(x, W2_ref[...], preferred_element_type=f32) + b2_ref[...]
    x = jnp.maximum(x, 0.0).astype(bf16)
    dh_ref[...] = jnp.dot(x, W3_ref[...], preferred_element_type=f32) + b3_ref[...]
    y = (jnp.dot(hr, We1rP_ref[...], preferred_element_type=f32)
         + jnp.dot(hc, We1cP_ref[...], preferred_element_type=f32)
         + jnp.dot(ea16, We1e_ref[...], preferred_element_type=f32)
         + be1_ref[...])
    y = jnp.maximum(y, 0.0).astype(bf16)
    eo_ref[...] = (ea + jnp.dot(y, We2_ref[...], preferred_element_type=f32)
                   + be2_ref[...])


def _edge_blk(i):
    return (i, 0)


def _full(i):
    return (0, 0)


def _tc_mlp(h_row, h_col, edge_attr, *weights):
    n_edges = h_row.shape[0]
    eb = EDGE_BLOCK
    in_specs = [
        pl.BlockSpec((eb, HALF_DIM), _edge_blk),
        pl.BlockSpec((eb, HALF_DIM), _edge_blk),
        pl.BlockSpec((eb, EDGE_DIM), _edge_blk),
    ] + [pl.BlockSpec(w.shape, _full) for w in weights]
    out_specs = (
        pl.BlockSpec((eb, NODE_DIM), _edge_blk),
        pl.BlockSpec((eb, EDGE_DIM), _edge_blk),
    )
    return pl.pallas_call(
        _mlp_body,
        grid=(n_edges // eb,),
        in_specs=in_specs,
        out_specs=out_specs,
        out_shape=(
            jax.ShapeDtypeStruct((n_edges, NODE_DIM), jnp.float32),
            jax.ShapeDtypeStruct((n_edges, EDGE_DIM), jnp.float32),
        ),
    )(h_row, h_col, edge_attr, *weights)


def kernel(h, edge_index, edge_attr, W1, b1, W2, b2, W3, b3,
           We1, be1, We2, be2):
    row = edge_index[0].astype(jnp.int32)
    col = edge_index[1].astype(jnp.int32)

    bf16 = jnp.bfloat16
    W1hP = W1[:NODE_DIM].astype(bf16)
    W1e = W1[NODE_DIM:].astype(bf16)
    W2 = W2.astype(bf16)
    W3 = W3.astype(bf16)
    We1rP = We1[:NODE_DIM].astype(bf16)
    We1cP = We1[NODE_DIM:2 * NODE_DIM].astype(bf16)
    We1e = We1[2 * NODE_DIM:].astype(bf16)
    We2 = We2.astype(bf16)
    b1r = b1.reshape(1, -1)
    b2r = b2.reshape(1, -1)
    b3r = b3.reshape(1, -1)
    be1r = be1.reshape(1, -1)
    be2r = be2.reshape(1, -1)

    u = jax.lax.bitcast_convert_type(h, jnp.uint32)

    def _rne(v):
        return (v + jnp.uint32(0x7FFF) + ((v >> 16) & jnp.uint32(1))) >> 16

    h_pack = jax.lax.bitcast_convert_type(
        _rne(u[:, :HALF_DIM]) | (_rne(u[:, HALF_DIM:]) << 16), jnp.int32)

    gather = _make_gather(SLICE_E)
    scatter = _make_scatter(SLICE_E)

    h_cur = h
    eo_parts = []
    for k in range(SLICES):
        sl = slice(k * SLICE_E, (k + 1) * SLICE_E)
        row_k = row[sl]
        col_k = col[sl]
        hr_k, hc_k = gather(h_pack, row_k, col_k)
        dh_k, eo_k = _tc_mlp(hr_k, hc_k, edge_attr[sl],
                             W1hP, W1e, b1r, W2, b2r, W3, b3r,
                             We1rP, We1cP, We1e, be1r, We2, be2r)
        eo_parts.append(eo_k)
        h_cur = scatter(h_cur, row_k, dh_k)

    edge_attr_new = jnp.concatenate(eo_parts, axis=0)
    return (h_cur, edge_attr_new)
